# baseline (speedup 1.0000x reference)
import functools

import jax
import jax.numpy as jnp
from jax import lax
from jax.experimental import pallas as pl
from jax.experimental.pallas import tpu as pltpu
from jax.experimental.pallas import tpu_sc as plsc

F32 = jnp.float32

NC = 2
NS = 16
L = 16
NW = NC * NS
CHUNK = 128


def _dense_proj_body(xa_ref, xp_ref, wa_ref, ba_ref, wp_ref, bp_ref,
                     as_ref, ad_ref,
                     xa_out, asrc_out, adst_out, ms_out, md_out):
    i = pl.program_id(0)
    xa = jnp.dot(xa_ref[...], wa_ref[...], preferred_element_type=F32) + ba_ref[...]
    xa_out[...] = xa
    s = jnp.dot(xa, as_ref[...], preferred_element_type=F32)
    asrc_out[...] = s
    xp = jnp.dot(xp_ref[...], wp_ref[...], preferred_element_type=F32) + bp_ref[...]
    t = jnp.dot(xp, ad_ref[...], preferred_element_type=F32)
    adst_out[...] = t

    sm = jnp.max(s, axis=0, keepdims=True)
    tm = jnp.max(t, axis=0, keepdims=True)

    @pl.when(i == 0)
    def _():
        ms_out[...] = jnp.full_like(ms_out, -1e30)
        md_out[...] = jnp.full_like(md_out, -1e30)

    ms_out[...] = jnp.maximum(ms_out[...], sm)
    md_out[...] = jnp.maximum(md_out[...], tm)


def _dense_proj(x_a, x_p, w_a, b_a, w_p, b_p, a_src, a_dst, bn):
    n, f = x_a.shape
    h8 = a_src.shape[1]
    grid = n // bn
    return pl.pallas_call(
        _dense_proj_body,
        grid=(grid,),
        in_specs=[
            pl.BlockSpec((bn, f), lambda i: (i, 0)),
            pl.BlockSpec((bn, f), lambda i: (i, 0)),
            pl.BlockSpec((f, f), lambda i: (0, 0)),
            pl.BlockSpec((1, f), lambda i: (0, 0)),
            pl.BlockSpec((f, f), lambda i: (0, 0)),
            pl.BlockSpec((1, f), lambda i: (0, 0)),
            pl.BlockSpec((f, h8), lambda i: (0, 0)),
            pl.BlockSpec((f, h8), lambda i: (0, 0)),
        ],
        out_specs=[
            pl.BlockSpec((bn, f), lambda i: (i, 0)),
            pl.BlockSpec((bn, h8), lambda i: (i, 0)),
            pl.BlockSpec((bn, h8), lambda i: (i, 0)),
            pl.BlockSpec((1, h8), lambda i: (0, 0)),
            pl.BlockSpec((1, h8), lambda i: (0, 0)),
        ],
        out_shape=[
            jax.ShapeDtypeStruct((n, f), F32),
            jax.ShapeDtypeStruct((n, h8), F32),
            jax.ShapeDtypeStruct((n, h8), F32),
            jax.ShapeDtypeStruct((1, h8), F32),
            jax.ShapeDtypeStruct((1, h8), F32),
        ],
    )(x_a, x_p, w_a, b_a, w_p, b_p, a_src, a_dst)


def _finish_body(p0_ref, p1_ref, w_ref, b_ref, y_ref):
    acc = jnp.maximum(p0_ref[0] + p1_ref[0], 0.0)
    y_ref[...] = jnp.dot(acc, w_ref[...], preferred_element_type=F32) + b_ref[...]


def _finish(parts, w, b, bn, n):
    f = parts.shape[2]
    out = w.shape[1]
    return pl.pallas_call(
        _finish_body,
        grid=(n // bn,),
        in_specs=[
            pl.BlockSpec((1, bn, f), lambda i: (0, i, 0)),
            pl.BlockSpec((1, bn, f), lambda i: (1, i, 0)),
            pl.BlockSpec((f, out), lambda i: (0, 0)),
            pl.BlockSpec((1, out), lambda i: (0, 0)),
        ],
        out_specs=pl.BlockSpec((bn, out), lambda i: (i, 0)),
        out_shape=jax.ShapeDtypeStruct((n, out), F32),
    )(parts, parts, w, b)


def _make_pass1(e_pad, n_pad, cpw0, cpw1):
    rpt = n_pad // NS
    mesh = plsc.VectorSubcoreMesh(core_axis_name="c", subcore_axis_name="s")

    @functools.partial(
        pl.kernel,
        out_type=(
            jax.ShapeDtypeStruct((e_pad, L), F32),
            jax.ShapeDtypeStruct((NC, n_pad, L), F32),
        ),
        mesh=mesh,
        compiler_params=pltpu.CompilerParams(use_tc_tiling_on_sc=False),
        scratch_types=[
            pltpu.VMEM((cpw0, CHUNK), jnp.int32),
            pltpu.VMEM((cpw0, CHUNK), jnp.int32),
            [pltpu.VMEM((CHUNK, L), F32)] * 4,
            [pltpu.VMEM((CHUNK, L), F32)] * 4,
            pltpu.VMEM((CHUNK, L), F32),
            pltpu.VMEM((CHUNK, L), F32),
            pltpu.VMEM((L,), F32),
            pltpu.VMEM_SHARED((n_pad, L), F32),
            [pltpu.SemaphoreType.DMA] * 4,
            [pltpu.SemaphoreType.DMA] * 4,
            pltpu.SemaphoreType.DMA,
            pltpu.SemaphoreType.DMA,
            pltpu.SemaphoreType.DMA,
            pltpu.SemaphoreType.DMA,
        ],
    )
    def pass1(src_hbm, dst_hbm, asrc_hbm, adst_hbm, m_hbm,
              e_out, s_out,
              src_v, dst_v, ga, gd, ev0, ev1, mv, s_sh,
              sa, sd, sc0, sc1, st0, st1):
        cid = lax.axis_index("c")
        sid = lax.axis_index("s")
        mycpw = jnp.where(cid == 0, cpw0, cpw1)
        cbase = jnp.where(cid == 0, sid * cpw0, NS * cpw0 + sid * cpw1)
        ev = (ev0, ev1)
        scat = (sc0, sc1)
        st = (st0, st1)

        @plsc.parallel_loop(0, CHUNK, unroll=8)
        def _(r):
            ev0[r, :] = jnp.zeros((L,), F32)

        @pl.loop(0, rpt // CHUNK)
        def _(k):
            pltpu.sync_copy(ev0, s_sh.at[pl.ds(sid * rpt + k * CHUNK, CHUNK)])

        plsc.subcore_barrier()

        pltpu.sync_copy(src_hbm.at[pl.ds(cbase, cpw0)], src_v)
        pltpu.sync_copy(dst_hbm.at[pl.ds(cbase, cpw0)], dst_v)
        pltpu.sync_copy(m_hbm, mv)
        m = mv[...]

        def issue(cc, g):
            pltpu.async_copy(asrc_hbm.at[src_v.at[cc]], ga[g], sa[g])
            pltpu.async_copy(adst_hbm.at[dst_v.at[cc]], gd[g], sd[g])

        def drain_stores(cc, b):
            pltpu.make_async_copy(ev[b], s_sh.at[dst_v.at[cc]], scat[b]).wait()
            pltpu.make_async_copy(
                ev[b], e_out.at[pl.ds((cbase + cc) * CHUNK, CHUNK)],
                st[b]).wait()

        issue(0, 0)
        issue(1, 1)
        issue(2, 2)

        @pl.loop(0, mycpw // 4)
        def _(quad):
            for slot in range(4):
                c = 4 * quad + slot
                b = slot & 1
                gslot = (slot + 3) & 3

                @pl.when(c + 3 < mycpw)
                def _():
                    issue(c + 3, gslot)

                pltpu.make_async_copy(asrc_hbm.at[src_v.at[c]], ga[slot],
                                      sa[slot]).wait()
                pltpu.make_async_copy(adst_hbm.at[dst_v.at[c]], gd[slot],
                                      sd[slot]).wait()

                @pl.when(c >= 2)
                def _():
                    drain_stores(c - 2, b)

                gab, gdb, evb = ga[slot], gd[slot], ev[b]

                @plsc.parallel_loop(0, CHUNK, unroll=8)
                def _(r):
                    al = gab[r, :] + gdb[r, :]
                    al = jnp.where(al >= 0.0, al, 0.2 * al) - m
                    evb[r, :] = jnp.exp(al)

                pltpu.async_copy(evb, s_sh.at[dst_v.at[c]], scat[b], add=True)
                pltpu.async_copy(
                    evb, e_out.at[pl.ds((cbase + c) * CHUNK, CHUNK)], st[b])

        drain_stores(mycpw - 2, 0)
        drain_stores(mycpw - 1, 1)

        plsc.subcore_barrier()
        pltpu.sync_copy(s_sh.at[pl.ds(sid * rpt, rpt)],
                        s_out.at[cid, pl.ds(sid * rpt, rpt)])

    return pass1


def _make_pass2(e_pad, n_pad, cpw0, cpw1, f):
    rpt = n_pad // NS
    mesh = plsc.VectorSubcoreMesh(core_axis_name="c", subcore_axis_name="s")
    nh = f // L

    @functools.partial(
        pl.kernel,
        out_type=jax.ShapeDtypeStruct((NC, n_pad, f), F32),
        mesh=mesh,
        compiler_params=pltpu.CompilerParams(use_tc_tiling_on_sc=False),
        scratch_types=[
            pltpu.VMEM((4, CHUNK), jnp.int32),
            pltpu.VMEM((4, CHUNK), jnp.int32),
            pltpu.VMEM((CHUNK, L), F32),
            pltpu.VMEM((CHUNK, L), F32),
            pltpu.VMEM((CHUNK, L), F32),
            pltpu.VMEM((CHUNK, L), F32),
            pltpu.VMEM((CHUNK, f), F32),
            pltpu.VMEM((CHUNK, f), F32),
            pltpu.VMEM_SHARED((n_pad, f), F32),
            pltpu.SemaphoreType.DMA,
            pltpu.SemaphoreType.DMA,
            pltpu.SemaphoreType.DMA,
            pltpu.SemaphoreType.DMA,
            pltpu.SemaphoreType.DMA,
            pltpu.SemaphoreType.DMA,
            pltpu.SemaphoreType.DMA,
            pltpu.SemaphoreType.DMA,
            [pltpu.SemaphoreType.DMA] * 4,
        ],
    )
    def pass2(src_hbm, dst_hbm, e_hbm, s_hbm, xa_hbm,
              out_parts,
              src_v, dst_v, ev0, ev1, sv0, sv1, xg0, xg1, out_sh,
              se0, se1, ss0, ss1, sx0, sx1, sc0, sc1, si):
        cid = lax.axis_index("c")
        sid = lax.axis_index("s")
        mycpw = jnp.where(cid == 0, cpw0, cpw1)
        cbase = jnp.where(cid == 0, sid * cpw0, NS * cpw0 + sid * cpw1)
        ev = (ev0, ev1)
        sv = (sv0, sv1)
        xg = (xg0, xg1)
        se = (se0, se1)
        ss = (ss0, ss1)
        sx = (sx0, sx1)
        scat = (sc0, sc1)

        with jax.named_scope("p2zero"):
            @plsc.parallel_loop(0, CHUNK, unroll=4)
            def _(r):
                for j in range(nh):
                    xg0[r, pl.ds(j * L, L)] = jnp.zeros((L,), F32)

            @pl.loop(0, rpt // CHUNK)
            def _(k):
                pltpu.sync_copy(xg0,
                                out_sh.at[pl.ds(sid * rpt + k * CHUNK, CHUNK)])

            plsc.subcore_barrier()

        def fetch_idx(cc, slot):
            pltpu.async_copy(src_hbm.at[cbase + cc], src_v.at[slot], si[slot])
            pltpu.async_copy(dst_hbm.at[cbase + cc], dst_v.at[slot], si[slot])

        def wait_idx(cc, slot):
            pltpu.make_async_copy(src_hbm.at[cbase + cc], src_v.at[slot],
                                  si[slot]).wait()
            pltpu.make_async_copy(dst_hbm.at[cbase + cc], dst_v.at[slot],
                                  si[slot]).wait()

        def issue(cc, b, slot):
            pltpu.async_copy(e_hbm.at[pl.ds((cbase + cc) * CHUNK, CHUNK)],
                             ev[b], se[b])
            pltpu.async_copy(s_hbm.at[dst_v.at[slot]], sv[b], ss[b])
            pltpu.async_copy(xa_hbm.at[src_v.at[slot]], xg[b], sx[b])

        fetch_idx(0, 0)
        fetch_idx(1, 1)
        wait_idx(0, 0)
        issue(0, 0, 0)

        @pl.loop(0, mycpw // 4)
        def _(quad):
            for slot in range(4):
                c = 4 * quad + slot
                b = slot & 1
                nslot = (slot + 1) & 3
                fslot = (slot + 2) & 3
                pslot = (slot - 1) & 3

                @pl.when(c + 2 < mycpw)
                def _():
                    fetch_idx(c + 2, fslot)

                @pl.when(c + 1 < mycpw)
                def _():
                    wait_idx(c + 1, nslot)
                    @pl.when(c >= 1)
                    def _():
                        pltpu.make_async_copy(
                            xg[1 - b], out_sh.at[dst_v.at[pslot]],
                            scat[1 - b]).wait()

                    issue(c + 1, 1 - b, nslot)

                pltpu.make_async_copy(
                    e_hbm.at[pl.ds((cbase + c) * CHUNK, CHUNK)],
                    ev[b], se[b]).wait()
                pltpu.make_async_copy(s_hbm.at[dst_v.at[slot]], sv[b],
                                      ss[b]).wait()
                pltpu.make_async_copy(xa_hbm.at[src_v.at[slot]], xg[b],
                                      sx[b]).wait()

                evb, svb, xgb = ev[b], sv[b], xg[b]

                @plsc.parallel_loop(0, CHUNK, unroll=4)
                def _(r):
                    a_row = evb[r, :] * svb[r, :]
                    for h in range(nh):
                        xgb[r, pl.ds(h * L, L)] = xgb[r, pl.ds(h * L, L)] * a_row[h]

                pltpu.async_copy(xgb, out_sh.at[dst_v.at[slot]], scat[b],
                                 add=True)

        pltpu.make_async_copy(xg[0], out_sh.at[dst_v.at[2]], scat[0]).wait()
        pltpu.make_async_copy(xg[1], out_sh.at[dst_v.at[3]], scat[1]).wait()

        plsc.subcore_barrier()

        with jax.named_scope("p2flush"):
            @pl.loop(0, rpt // CHUNK)
            def _(k):
                row0 = sid * rpt + k * CHUNK
                pltpu.sync_copy(out_sh.at[pl.ds(row0, CHUNK)],
                                out_parts.at[cid, pl.ds(row0, CHUNK)])

    return pass2


def kernel(x_author, x_paper, ei_writes, ei_rev, W_author, b_author, W_paper,
           b_paper, att_src_writes, att_dst_writes, att_src_rev, att_dst_rev,
           k_lin_W, k_lin_b, q, lin_W, lin_b):
    n, f = x_author.shape
    heads, dh = att_src_writes.shape
    e = ei_writes.shape[1]

    tot = 4 * (-(-e // (NS * CHUNK * 4)))
    p1c0 = max(4, 4 * round(0.65 * tot / 4))
    p1c1 = tot - p1c0
    p2c0 = max(4, 4 * round(0.775 * tot / 4))
    p2c1 = tot - p2c0
    e_pad = NS * tot * CHUNK
    n_pad = -(-(n + 1) // (NS * CHUNK)) * (NS * CHUNK)

    eye = jnp.eye(heads, dtype=F32)
    a_src = (att_src_writes[:, :, None] * eye[:, None, :]).reshape(f, heads)
    a_dst = (att_dst_writes[:, :, None] * eye[:, None, :]).reshape(f, heads)

    xa, asrc, adst, ms, md = _dense_proj(
        x_author, x_paper, W_author, b_author.reshape(1, f),
        W_paper, b_paper.reshape(1, f), a_src, a_dst, bn=1000)

    t = ms + md
    m16 = jnp.concatenate(
        [jnp.where(t >= 0.0, t, 0.2 * t).reshape(heads),
         jnp.zeros((L - heads,), F32)])

    pad = n_pad - n
    asrc_t = jnp.concatenate([
        jnp.concatenate([asrc, jnp.zeros((n, L - heads), F32)], axis=1),
        jnp.concatenate([jnp.full((pad, heads), -1e30, F32),
                         jnp.zeros((pad, L - heads), F32)], axis=1)], axis=0)
    adst_t = jnp.concatenate([
        jnp.concatenate([adst, jnp.zeros((n, L - heads), F32)], axis=1),
        jnp.zeros((pad, L), F32)], axis=0)
    xa_t = jnp.concatenate([xa, jnp.zeros((pad, f), F32)], axis=0)

    extra = (2 * p1c0 - tot) * CHUNK if p1c0 > p1c1 else 0
    fill = jnp.full((e_pad + extra - e,), n, jnp.int32)
    src2 = jnp.concatenate([ei_writes[0], fill]).reshape(-1, CHUNK)
    dst2 = jnp.concatenate([ei_writes[1], fill]).reshape(-1, CHUNK)

    pass1 = _make_pass1(e_pad, n_pad, p1c0, p1c1)
    e_vals, s_parts = pass1(src2, dst2, asrc_t, adst_t, m16)
    s_t = 1.0 / (s_parts[0] + s_parts[1] + 1e-16)

    pass2 = _make_pass2(e_pad, n_pad, p2c0, p2c1, f)
    out_parts = pass2(src2, dst2, e_vals, s_t, xa_t)

    y = _finish(out_parts, lin_W, lin_b.reshape(1, -1), bn=1000, n=n)
    return y


# speedup vs baseline: 1.2194x; 1.2194x over previous
import functools

import jax
import jax.numpy as jnp
from jax import lax
from jax.experimental import pallas as pl
from jax.experimental.pallas import tpu as pltpu
from jax.experimental.pallas import tpu_sc as plsc

F32 = jnp.float32

NC = 2
NS = 16
L = 16
NW = NC * NS
CHUNK = 128


def _dense_proj_body(xa_ref, xp_ref, wa_ref, ba_ref, wp_ref, bp_ref,
                     as_ref, ad_ref,
                     xa_out, asrc_out, adst_out, ms_out, md_out):
    i = pl.program_id(0)
    xa = jnp.dot(xa_ref[...], wa_ref[...], preferred_element_type=F32) + ba_ref[...]
    xa_out[...] = xa
    s = jnp.dot(xa, as_ref[...], preferred_element_type=F32)
    asrc_out[...] = s
    xp = jnp.dot(xp_ref[...], wp_ref[...], preferred_element_type=F32) + bp_ref[...]
    t = jnp.dot(xp, ad_ref[...], preferred_element_type=F32)
    adst_out[...] = t

    sm = jnp.max(s, axis=0, keepdims=True)
    tm = jnp.max(t, axis=0, keepdims=True)

    @pl.when(i == 0)
    def _():
        ms_out[...] = jnp.full_like(ms_out, -1e30)
        md_out[...] = jnp.full_like(md_out, -1e30)

    ms_out[...] = jnp.maximum(ms_out[...], sm)
    md_out[...] = jnp.maximum(md_out[...], tm)


def _dense_proj(x_a, x_p, w_a, b_a, w_p, b_p, a_src, a_dst, bn):
    n, f = x_a.shape
    h8 = a_src.shape[1]
    grid = n // bn
    return pl.pallas_call(
        _dense_proj_body,
        grid=(grid,),
        in_specs=[
            pl.BlockSpec((bn, f), lambda i: (i, 0)),
            pl.BlockSpec((bn, f), lambda i: (i, 0)),
            pl.BlockSpec((f, f), lambda i: (0, 0)),
            pl.BlockSpec((1, f), lambda i: (0, 0)),
            pl.BlockSpec((f, f), lambda i: (0, 0)),
            pl.BlockSpec((1, f), lambda i: (0, 0)),
            pl.BlockSpec((f, h8), lambda i: (0, 0)),
            pl.BlockSpec((f, h8), lambda i: (0, 0)),
        ],
        out_specs=[
            pl.BlockSpec((bn, f), lambda i: (i, 0)),
            pl.BlockSpec((bn, h8), lambda i: (i, 0)),
            pl.BlockSpec((bn, h8), lambda i: (i, 0)),
            pl.BlockSpec((1, h8), lambda i: (0, 0)),
            pl.BlockSpec((1, h8), lambda i: (0, 0)),
        ],
        out_shape=[
            jax.ShapeDtypeStruct((n, f), F32),
            jax.ShapeDtypeStruct((n, h8), F32),
            jax.ShapeDtypeStruct((n, h8), F32),
            jax.ShapeDtypeStruct((1, h8), F32),
            jax.ShapeDtypeStruct((1, h8), F32),
        ],
    )(x_a, x_p, w_a, b_a, w_p, b_p, a_src, a_dst)


def _finish_body(p0_ref, p1_ref, w_ref, b_ref, y_ref):
    acc = jnp.maximum(p0_ref[0] + p1_ref[0], 0.0)
    y_ref[...] = jnp.dot(acc, w_ref[...], preferred_element_type=F32) + b_ref[...]


def _finish(parts, w, b, bn, n):
    f = parts.shape[2]
    out = w.shape[1]
    return pl.pallas_call(
        _finish_body,
        grid=(n // bn,),
        in_specs=[
            pl.BlockSpec((1, bn, f), lambda i: (0, i, 0)),
            pl.BlockSpec((1, bn, f), lambda i: (1, i, 0)),
            pl.BlockSpec((f, out), lambda i: (0, 0)),
            pl.BlockSpec((1, out), lambda i: (0, 0)),
        ],
        out_specs=pl.BlockSpec((bn, out), lambda i: (i, 0)),
        out_shape=jax.ShapeDtypeStruct((n, out), F32),
    )(parts, parts, w, b)


def _make_pass1(e_pad, n_pad, cpw0, cpw1):
    rpt = n_pad // NS
    mesh = plsc.VectorSubcoreMesh(core_axis_name="c", subcore_axis_name="s")

    @functools.partial(
        pl.kernel,
        out_type=(
            jax.ShapeDtypeStruct((e_pad, L), F32),
            jax.ShapeDtypeStruct((NC, n_pad, L), F32),
        ),
        mesh=mesh,
        compiler_params=pltpu.CompilerParams(use_tc_tiling_on_sc=False),
        scratch_types=[
            pltpu.VMEM((cpw0, CHUNK), jnp.int32),
            pltpu.VMEM((cpw0, CHUNK), jnp.int32),
            [pltpu.VMEM((CHUNK, L), F32)] * 4,
            [pltpu.VMEM((CHUNK, L), F32)] * 4,
            pltpu.VMEM((CHUNK, L), F32),
            pltpu.VMEM((CHUNK, L), F32),
            pltpu.VMEM((L,), F32),
            pltpu.VMEM_SHARED((n_pad, L), F32),
            [pltpu.SemaphoreType.DMA] * 4,
            [pltpu.SemaphoreType.DMA] * 4,
            pltpu.SemaphoreType.DMA,
            pltpu.SemaphoreType.DMA,
            pltpu.SemaphoreType.DMA,
            pltpu.SemaphoreType.DMA,
        ],
    )
    def pass1(src_hbm, dst_hbm, asrc_hbm, adst_hbm, m_hbm,
              e_out, s_out,
              src_v, dst_v, ga, gd, ev0, ev1, mv, s_sh,
              sa, sd, sc0, sc1, st0, st1):
        cid = lax.axis_index("c")
        sid = lax.axis_index("s")
        mycpw = jnp.where(cid == 0, cpw0, cpw1)
        cbase = jnp.where(cid == 0, sid * cpw0, NS * cpw0 + sid * cpw1)
        ev = (ev0, ev1)
        scat = (sc0, sc1)
        st = (st0, st1)

        @plsc.parallel_loop(0, CHUNK, unroll=8)
        def _(r):
            ev0[r, :] = jnp.zeros((L,), F32)

        @pl.loop(0, rpt // CHUNK)
        def _(k):
            pltpu.sync_copy(ev0, s_sh.at[pl.ds(sid * rpt + k * CHUNK, CHUNK)])

        plsc.subcore_barrier()

        pltpu.sync_copy(src_hbm.at[pl.ds(cbase, cpw0)], src_v)
        pltpu.sync_copy(dst_hbm.at[pl.ds(cbase, cpw0)], dst_v)
        pltpu.sync_copy(m_hbm, mv)
        m = mv[...]

        def issue(cc, g):
            pltpu.async_copy(asrc_hbm.at[src_v.at[cc]], ga[g], sa[g])
            pltpu.async_copy(adst_hbm.at[dst_v.at[cc]], gd[g], sd[g])

        def drain_stores(cc, b):
            pltpu.make_async_copy(ev[b], s_sh.at[dst_v.at[cc]], scat[b]).wait()
            pltpu.make_async_copy(
                ev[b], e_out.at[pl.ds((cbase + cc) * CHUNK, CHUNK)],
                st[b]).wait()

        issue(0, 0)
        issue(1, 1)
        issue(2, 2)

        @pl.loop(0, mycpw // 4)
        def _(quad):
            for slot in range(4):
                c = 4 * quad + slot
                b = slot & 1
                gslot = (slot + 3) & 3

                @pl.when(c + 3 < mycpw)
                def _():
                    issue(c + 3, gslot)

                pltpu.make_async_copy(asrc_hbm.at[src_v.at[c]], ga[slot],
                                      sa[slot]).wait()
                pltpu.make_async_copy(adst_hbm.at[dst_v.at[c]], gd[slot],
                                      sd[slot]).wait()

                @pl.when(c >= 2)
                def _():
                    drain_stores(c - 2, b)

                gab, gdb, evb = ga[slot], gd[slot], ev[b]

                @plsc.parallel_loop(0, CHUNK, unroll=8)
                def _(r):
                    al = gab[r, :] + gdb[r, :]
                    al = jnp.where(al >= 0.0, al, 0.2 * al) - m
                    evb[r, :] = jnp.exp(al)

                pltpu.async_copy(evb, s_sh.at[dst_v.at[c]], scat[b], add=True)
                pltpu.async_copy(
                    evb, e_out.at[pl.ds((cbase + c) * CHUNK, CHUNK)], st[b])

        drain_stores(mycpw - 2, 0)
        drain_stores(mycpw - 1, 1)

        plsc.subcore_barrier()
        pltpu.sync_copy(s_sh.at[pl.ds(sid * rpt, rpt)],
                        s_out.at[cid, pl.ds(sid * rpt, rpt)])

    return pass1


def _make_pass2(e_pad, n_pad, cpw0, cpw1, f):
    rpt = n_pad // NS
    mesh = plsc.VectorSubcoreMesh(core_axis_name="c", subcore_axis_name="s")
    nh = f // L

    @functools.partial(
        pl.kernel,
        out_type=jax.ShapeDtypeStruct((NC, n_pad, f), F32),
        mesh=mesh,
        compiler_params=pltpu.CompilerParams(use_tc_tiling_on_sc=False),
        scratch_types=[
            pltpu.VMEM((4, CHUNK), jnp.int32),
            pltpu.VMEM((4, CHUNK), jnp.int32),
            pltpu.VMEM((CHUNK, L), F32),
            pltpu.VMEM((CHUNK, L), F32),
            pltpu.VMEM((CHUNK, L), F32),
            pltpu.VMEM((CHUNK, L), F32),
            pltpu.VMEM((CHUNK, f), F32),
            pltpu.VMEM((CHUNK, f), F32),
            pltpu.VMEM_SHARED((n_pad, f), F32),
            pltpu.SemaphoreType.DMA,
            pltpu.SemaphoreType.DMA,
            pltpu.SemaphoreType.DMA,
            pltpu.SemaphoreType.DMA,
            pltpu.SemaphoreType.DMA,
            pltpu.SemaphoreType.DMA,
            pltpu.SemaphoreType.DMA,
            pltpu.SemaphoreType.DMA,
            [pltpu.SemaphoreType.DMA] * 4,
        ],
    )
    def pass2(src_hbm, dst_hbm, e_hbm, s_hbm, xa_hbm,
              out_parts,
              src_v, dst_v, ev0, ev1, sv0, sv1, xg0, xg1, out_sh,
              se0, se1, ss0, ss1, sx0, sx1, sc0, sc1, si):
        cid = lax.axis_index("c")
        sid = lax.axis_index("s")
        mycpw = jnp.where(cid == 0, cpw0, cpw1)
        cbase = jnp.where(cid == 0, sid * cpw0, NS * cpw0 + sid * cpw1)
        ev = (ev0, ev1)
        sv = (sv0, sv1)
        xg = (xg0, xg1)
        se = (se0, se1)
        ss = (ss0, ss1)
        sx = (sx0, sx1)
        scat = (sc0, sc1)

        with jax.named_scope("p2zero"):
            @plsc.parallel_loop(0, CHUNK, unroll=4)
            def _(r):
                for j in range(nh):
                    xg0[r, pl.ds(j * L, L)] = jnp.zeros((L,), F32)

            @pl.loop(0, rpt // CHUNK)
            def _(k):
                pltpu.sync_copy(xg0,
                                out_sh.at[pl.ds(sid * rpt + k * CHUNK, CHUNK)])

            plsc.subcore_barrier()

        def fetch_idx(cc, slot):
            pltpu.async_copy(src_hbm.at[cbase + cc], src_v.at[slot], si[slot])
            pltpu.async_copy(dst_hbm.at[cbase + cc], dst_v.at[slot], si[slot])

        def wait_idx(cc, slot):
            pltpu.make_async_copy(src_hbm.at[cbase + cc], src_v.at[slot],
                                  si[slot]).wait()
            pltpu.make_async_copy(dst_hbm.at[cbase + cc], dst_v.at[slot],
                                  si[slot]).wait()

        def issue(cc, b, slot):
            pltpu.async_copy(e_hbm.at[pl.ds((cbase + cc) * CHUNK, CHUNK)],
                             ev[b], se[b])
            pltpu.async_copy(s_hbm.at[dst_v.at[slot]], sv[b], ss[b])
            pltpu.async_copy(xa_hbm.at[src_v.at[slot]], xg[b], sx[b])

        fetch_idx(0, 0)
        fetch_idx(1, 1)
        wait_idx(0, 0)
        issue(0, 0, 0)

        @pl.loop(0, mycpw // 4)
        def _(quad):
            for slot in range(4):
                c = 4 * quad + slot
                b = slot & 1
                nslot = (slot + 1) & 3
                fslot = (slot + 2) & 3
                pslot = (slot - 1) & 3

                @pl.when(c + 2 < mycpw)
                def _():
                    fetch_idx(c + 2, fslot)

                @pl.when(c + 1 < mycpw)
                def _():
                    wait_idx(c + 1, nslot)
                    @pl.when(c >= 1)
                    def _():
                        pltpu.make_async_copy(
                            xg[1 - b], out_sh.at[dst_v.at[pslot]],
                            scat[1 - b]).wait()

                    issue(c + 1, 1 - b, nslot)

                pltpu.make_async_copy(
                    e_hbm.at[pl.ds((cbase + c) * CHUNK, CHUNK)],
                    ev[b], se[b]).wait()
                pltpu.make_async_copy(s_hbm.at[dst_v.at[slot]], sv[b],
                                      ss[b]).wait()
                pltpu.make_async_copy(xa_hbm.at[src_v.at[slot]], xg[b],
                                      sx[b]).wait()

                evb, svb, xgb = ev[b], sv[b], xg[b]

                @plsc.parallel_loop(0, CHUNK, unroll=4)
                def _(r):
                    a_row = evb[r, :] * svb[r, :]
                    for h in range(nh):
                        xgb[r, pl.ds(h * L, L)] = xgb[r, pl.ds(h * L, L)] * a_row[h]

                pltpu.async_copy(xgb, out_sh.at[dst_v.at[slot]], scat[b],
                                 add=True)

        pltpu.make_async_copy(xg[0], out_sh.at[dst_v.at[2]], scat[0]).wait()
        pltpu.make_async_copy(xg[1], out_sh.at[dst_v.at[3]], scat[1]).wait()

        plsc.subcore_barrier()

        with jax.named_scope("p2flush"):
            @pl.loop(0, rpt // CHUNK)
            def _(k):
                row0 = sid * rpt + k * CHUNK
                pltpu.sync_copy(out_sh.at[pl.ds(row0, CHUNK)],
                                out_parts.at[cid, pl.ds(row0, CHUNK)])

    return pass2


def kernel(x_author, x_paper, ei_writes, ei_rev, W_author, b_author, W_paper,
           b_paper, att_src_writes, att_dst_writes, att_src_rev, att_dst_rev,
           k_lin_W, k_lin_b, q, lin_W, lin_b):
    n, f = x_author.shape
    heads, dh = att_src_writes.shape
    e = ei_writes.shape[1]

    tot = 4 * (-(-e // (NS * CHUNK * 4)))
    p1c0 = max(4, 4 * round(0.75 * tot / 4))
    p1c1 = tot - p1c0
    p2c0 = max(4, 4 * round(0.9 * tot / 4))
    p2c1 = tot - p2c0
    e_pad = NS * tot * CHUNK
    n_pad = -(-(n + 1) // (NS * CHUNK)) * (NS * CHUNK)

    eye = jnp.eye(heads, dtype=F32)
    a_src = (att_src_writes[:, :, None] * eye[:, None, :]).reshape(f, heads)
    a_dst = (att_dst_writes[:, :, None] * eye[:, None, :]).reshape(f, heads)

    xa, asrc, adst, ms, md = _dense_proj(
        x_author, x_paper, W_author, b_author.reshape(1, f),
        W_paper, b_paper.reshape(1, f), a_src, a_dst, bn=1000)

    t = ms + md
    m16 = jnp.concatenate(
        [jnp.where(t >= 0.0, t, 0.2 * t).reshape(heads),
         jnp.zeros((L - heads,), F32)])

    pad = n_pad - n
    asrc_t = jnp.concatenate([
        jnp.concatenate([asrc, jnp.zeros((n, L - heads), F32)], axis=1),
        jnp.concatenate([jnp.full((pad, heads), -1e30, F32),
                         jnp.zeros((pad, L - heads), F32)], axis=1)], axis=0)
    adst_t = jnp.concatenate([
        jnp.concatenate([adst, jnp.zeros((n, L - heads), F32)], axis=1),
        jnp.zeros((pad, L), F32)], axis=0)
    xa_t = jnp.concatenate([xa, jnp.zeros((pad, f), F32)], axis=0)

    extra = (2 * p1c0 - tot) * CHUNK if p1c0 > p1c1 else 0
    fill = jnp.full((e_pad + extra - e,), n, jnp.int32)
    src2 = jnp.concatenate([ei_writes[0], fill]).reshape(-1, CHUNK)
    dst2 = jnp.concatenate([ei_writes[1], fill]).reshape(-1, CHUNK)

    pass1 = _make_pass1(e_pad, n_pad, p1c0, p1c1)
    e_vals, s_parts = pass1(src2, dst2, asrc_t, adst_t, m16)
    s_t = 1.0 / (s_parts[0] + s_parts[1] + 1e-16)

    pass2 = _make_pass2(e_pad, n_pad, p2c0, p2c1, f)
    out_parts = pass2(src2, dst2, e_vals, s_t, xa_t)

    y = _finish(out_parts, lin_W, lin_b.reshape(1, -1), bn=1000, n=n)
    return y
